# baseline (speedup 1.0000x reference)
import functools

import jax
import jax.numpy as jnp
from jax import lax
from jax.experimental import pallas as pl
from jax.experimental.pallas import tpu as pltpu
from jax.experimental.pallas import tpu_sc as plsc

NC = 2
NS = 16
NW = NC * NS

QUARTER = 16384


def _combine_body(sym_ref, pos_ref, t_ref):
    t_ref[...] = pos_ref[...] + sym_ref[...]


def _combine_tables(sym_padded, pos):
    return pl.pallas_call(
        _combine_body,
        out_shape=jax.ShapeDtypeStruct(pos.shape, pos.dtype),
    )(sym_padded, pos)


def _sc_embed(tt, idx1, dist1, b, l, d):
    v = tt.shape[1]
    b_per_w = b // NW

    mesh = plsc.VectorSubcoreMesh(
        core_axis_name="c", subcore_axis_name="s", num_cores=NC,
        num_subcores=NS)

    @functools.partial(
        pl.kernel,
        out_type=jax.ShapeDtypeStruct((b * l * d,), jnp.float32),
        mesh=mesh,
        compiler_params=pltpu.CompilerParams(needs_layout_passes=False,
                                             use_tc_tiling_on_sc=False),
        scratch_types=[
            pltpu.VMEM((d, v), jnp.float32),
            pltpu.VMEM((2, l), jnp.int32),
            pltpu.VMEM((2, l), jnp.float32),
            pltpu.VMEM((2, QUARTER), jnp.float32),
            pltpu.SemaphoreType.DMA,
            pltpu.SemaphoreType.DMA,
            pltpu.SemaphoreType.DMA,
        ],
    )
    def run(tt_hbm, idx_hbm, dist_hbm, out_hbm, tt_v, idx_v, dist_v, buf_v,
            in_sem, o_sem0, o_sem1):
        wid = lax.axis_index("s") * NC + lax.axis_index("c")
        base = wid * b_per_w
        o_sems = (o_sem0, o_sem1)

        pltpu.sync_copy(tt_hbm, tt_v)

        def start_fetch(bi, s):
            off = pl.multiple_of((base + bi) * l, l)
            pltpu.async_copy(idx_hbm.at[pl.ds(off, l)], idx_v.at[s], in_sem)
            pltpu.async_copy(dist_hbm.at[pl.ds(off, l)], dist_v.at[s],
                             in_sem)

        def wait_fetch(s):
            pltpu.make_async_copy(idx_hbm.at[pl.ds(0, l)], idx_v.at[s],
                                  in_sem).wait()
            pltpu.make_async_copy(dist_hbm.at[pl.ds(0, l)], dist_v.at[s],
                                  in_sem).wait()

        def quarter_compute(s, q, bslot):
            d_base = q * 16
            with_dist = (d_base + 16 == d)

            @plsc.parallel_loop(0, l // 16, step=1, unroll=1)
            def j_body(j):
                jo = pl.multiple_of(j * 16, 16)
                idx_vec = idx_v[s, pl.ds(jo, 16)]
                lt = j // 8
                k = j - lt * 8
                base_j = lt * 1024 + k * 16
                for d8l in range(16):
                    dd = d_base + d8l
                    val = plsc.load_gather(
                        tt_v, [jnp.full((16,), dd, jnp.int32), idx_vec])
                    if with_dist and d8l == 15:
                        val = val + dist_v[s, pl.ds(jo, 16)]
                    boff = pl.multiple_of(
                        base_j + (d8l // 8) * 8192 + (d8l % 8) * 128, 16)
                    buf_v[bslot, pl.ds(boff, 16)] = val

        def start_out(bi, q, bslot):
            off = pl.multiple_of((base + bi) * (l * d) + q * QUARTER, QUARTER)
            pltpu.async_copy(buf_v.at[bslot],
                             out_hbm.at[pl.ds(off, QUARTER)], o_sems[bslot])

        def wait_out(bslot):
            pltpu.make_async_copy(buf_v.at[bslot],
                                  out_hbm.at[pl.ds(0, QUARTER)],
                                  o_sems[bslot]).wait()

        n_q = d // 16

        def b_body(i, _):
            for bb in range(2):
                bi = 2 * i + bb
                wait_fetch(bb)

                @pl.when(bi + 1 < b_per_w)
                def _():
                    start_fetch(bi + 1, 1 - bb)

                for q in range(n_q):
                    step = bb * n_q + q
                    bslot = step % 2
                    if step >= 2:
                        wait_out(bslot)
                    else:
                        @pl.when(i > 0)
                        def _():
                            wait_out(bslot)

                    quarter_compute(bb, q, bslot)
                    start_out(bi, q, bslot)
            return 0

        start_fetch(0, 0)
        lax.fori_loop(0, b_per_w // 2, b_body, 0)
        wait_out(0)
        wait_out(1)

    return run(tt, idx1, dist1)


def kernel(used_symbols, distribution, symbol_embeddings, positional_embeddings):
    b, l = used_symbols.shape
    v, dm1 = symbol_embeddings.shape
    d = dm1 + 1

    sym_padded = jnp.pad(symbol_embeddings, ((0, 0), (0, 1)))
    table = _combine_tables(sym_padded, positional_embeddings)
    tt = table.T

    idx1 = used_symbols.astype(jnp.int32).reshape(b * l)
    dist1 = distribution.reshape(b * l)
    flat = _sc_embed(tt, idx1, dist1, b, l, d)

    out5 = flat.reshape(b, d // 8, l // 128, 8, 128)
    return out5.transpose(0, 2, 4, 1, 3).reshape(b, l, d)


# speedup vs baseline: 1.0073x; 1.0073x over previous
import functools

import jax
import jax.numpy as jnp
from jax import lax
from jax.experimental import pallas as pl
from jax.experimental.pallas import tpu as pltpu
from jax.experimental.pallas import tpu_sc as plsc

NC = 2
NS = 16
NW = NC * NS

QUARTER = 16384


def _combine_body(sym_ref, pos_ref, t_ref):
    t_ref[...] = pos_ref[...] + sym_ref[...]


def _combine_tables(sym_padded, pos):
    return pl.pallas_call(
        _combine_body,
        out_shape=jax.ShapeDtypeStruct(pos.shape, pos.dtype),
    )(sym_padded, pos)


def _sc_embed(tt, idx1, dist1, b, l, d):
    v = tt.shape[1]
    b_per_w = b // NW

    mesh = plsc.VectorSubcoreMesh(
        core_axis_name="c", subcore_axis_name="s", num_cores=NC,
        num_subcores=NS)

    @functools.partial(
        pl.kernel,
        out_type=jax.ShapeDtypeStruct((b * l * d,), jnp.float32),
        mesh=mesh,
        compiler_params=pltpu.CompilerParams(needs_layout_passes=False,
                                             use_tc_tiling_on_sc=False),
        scratch_types=[
            pltpu.VMEM((d, v), jnp.float32),
            pltpu.VMEM((2, l), jnp.int32),
            pltpu.VMEM((2, l), jnp.float32),
            pltpu.VMEM((2, QUARTER), jnp.float32),
            pltpu.SemaphoreType.DMA,
            pltpu.SemaphoreType.DMA,
            pltpu.SemaphoreType.DMA,
        ],
    )
    def run(tt_hbm, idx_hbm, dist_hbm, out_hbm, tt_v, idx_v, dist_v, buf_v,
            in_sem, o_sem0, o_sem1):
        wid = lax.axis_index("s") * NC + lax.axis_index("c")
        base = wid * b_per_w
        o_sems = (o_sem0, o_sem1)

        pltpu.sync_copy(tt_hbm, tt_v)

        def start_fetch(bi, s):
            off = pl.multiple_of((base + bi) * l, l)
            pltpu.async_copy(idx_hbm.at[pl.ds(off, l)], idx_v.at[s], in_sem)
            pltpu.async_copy(dist_hbm.at[pl.ds(off, l)], dist_v.at[s],
                             in_sem)

        def wait_fetch(s):
            pltpu.make_async_copy(idx_hbm.at[pl.ds(0, l)], idx_v.at[s],
                                  in_sem).wait()
            pltpu.make_async_copy(dist_hbm.at[pl.ds(0, l)], dist_v.at[s],
                                  in_sem).wait()

        def quarter_compute(s, q, bslot):
            d_base = q * 16
            with_dist = (d_base + 16 == d)

            @plsc.parallel_loop(0, l // 16, step=1, unroll=2)
            def j_body(j):
                jo = pl.multiple_of(j * 16, 16)
                idx_vec = idx_v[s, pl.ds(jo, 16)]
                lt = j // 8
                k = j - lt * 8
                base_j = lt * 1024 + k * 16
                for d8l in range(16):
                    dd = d_base + d8l
                    val = plsc.load_gather(
                        tt_v, [jnp.full((16,), dd, jnp.int32), idx_vec])
                    if with_dist and d8l == 15:
                        val = val + dist_v[s, pl.ds(jo, 16)]
                    boff = pl.multiple_of(
                        base_j + (d8l // 8) * 8192 + (d8l % 8) * 128, 16)
                    buf_v[bslot, pl.ds(boff, 16)] = val

        def start_out(bi, q, bslot):
            off = pl.multiple_of((base + bi) * (l * d) + q * QUARTER, QUARTER)
            pltpu.async_copy(buf_v.at[bslot],
                             out_hbm.at[pl.ds(off, QUARTER)], o_sems[bslot])

        def wait_out(bslot):
            pltpu.make_async_copy(buf_v.at[bslot],
                                  out_hbm.at[pl.ds(0, QUARTER)],
                                  o_sems[bslot]).wait()

        n_q = d // 16

        def b_body(i, _):
            for bb in range(2):
                bi = 2 * i + bb
                wait_fetch(bb)

                @pl.when(bi + 1 < b_per_w)
                def _():
                    start_fetch(bi + 1, 1 - bb)

                for q in range(n_q):
                    step = bb * n_q + q
                    bslot = step % 2
                    if step >= 2:
                        wait_out(bslot)
                    else:
                        @pl.when(i > 0)
                        def _():
                            wait_out(bslot)

                    quarter_compute(bb, q, bslot)
                    start_out(bi, q, bslot)
            return 0

        start_fetch(0, 0)
        lax.fori_loop(0, b_per_w // 2, b_body, 0)
        wait_out(0)
        wait_out(1)

    return run(tt, idx1, dist1)


def kernel(used_symbols, distribution, symbol_embeddings, positional_embeddings):
    b, l = used_symbols.shape
    v, dm1 = symbol_embeddings.shape
    d = dm1 + 1

    sym_padded = jnp.pad(symbol_embeddings, ((0, 0), (0, 1)))
    table = _combine_tables(sym_padded, positional_embeddings)
    tt = table.T

    idx1 = used_symbols.astype(jnp.int32).reshape(b * l)
    dist1 = distribution.reshape(b * l)
    flat = _sc_embed(tt, idx1, dist1, b, l, d)

    out5 = flat.reshape(b, d // 8, l // 128, 8, 128)
    return out5.transpose(0, 2, 4, 1, 3).reshape(b, l, d)


# speedup vs baseline: 1.0092x; 1.0019x over previous
import functools

import jax
import jax.numpy as jnp
from jax import lax
from jax.experimental import pallas as pl
from jax.experimental.pallas import tpu as pltpu
from jax.experimental.pallas import tpu_sc as plsc

NC = 2
NS = 16
NW = NC * NS

QUARTER = 16384


def _combine_body(sym_ref, pos_ref, t_ref):
    t_ref[...] = jnp.transpose(pos_ref[...] + sym_ref[...])


def _combine_tables(sym_padded, pos):
    return pl.pallas_call(
        _combine_body,
        out_shape=jax.ShapeDtypeStruct(pos.shape[::-1], pos.dtype),
    )(sym_padded, pos)


def _sc_embed(tt, idx1, dist1, b, l, d):
    v = tt.shape[1]
    b_per_w = b // NW

    mesh = plsc.VectorSubcoreMesh(
        core_axis_name="c", subcore_axis_name="s", num_cores=NC,
        num_subcores=NS)

    @functools.partial(
        pl.kernel,
        out_type=jax.ShapeDtypeStruct((b * l * d,), jnp.float32),
        mesh=mesh,
        compiler_params=pltpu.CompilerParams(needs_layout_passes=False,
                                             use_tc_tiling_on_sc=False),
        scratch_types=[
            pltpu.VMEM((d, v), jnp.float32),
            pltpu.VMEM((2, l), jnp.int32),
            pltpu.VMEM((2, l), jnp.float32),
            pltpu.VMEM((2, QUARTER), jnp.float32),
            pltpu.SemaphoreType.DMA,
            pltpu.SemaphoreType.DMA,
            pltpu.SemaphoreType.DMA,
        ],
    )
    def run(tt_hbm, idx_hbm, dist_hbm, out_hbm, tt_v, idx_v, dist_v, buf_v,
            in_sem, o_sem0, o_sem1):
        wid = lax.axis_index("s") * NC + lax.axis_index("c")
        base = wid * b_per_w
        o_sems = (o_sem0, o_sem1)

        pltpu.sync_copy(tt_hbm, tt_v)

        def start_fetch(bi, s):
            off = pl.multiple_of((base + bi) * l, l)
            pltpu.async_copy(idx_hbm.at[pl.ds(off, l)], idx_v.at[s], in_sem)
            pltpu.async_copy(dist_hbm.at[pl.ds(off, l)], dist_v.at[s],
                             in_sem)

        def wait_fetch(s):
            pltpu.make_async_copy(idx_hbm.at[pl.ds(0, l)], idx_v.at[s],
                                  in_sem).wait()
            pltpu.make_async_copy(dist_hbm.at[pl.ds(0, l)], dist_v.at[s],
                                  in_sem).wait()

        def quarter_compute(s, q, bslot):
            d_base = q * 16
            with_dist = (d_base + 16 == d)

            @plsc.parallel_loop(0, l // 16, step=1, unroll=2)
            def j_body(j):
                jo = pl.multiple_of(j * 16, 16)
                idx_vec = idx_v[s, pl.ds(jo, 16)]
                lt = j // 8
                k = j - lt * 8
                base_j = lt * 1024 + k * 16
                for d8l in range(16):
                    dd = d_base + d8l
                    val = plsc.load_gather(
                        tt_v, [jnp.full((16,), dd, jnp.int32), idx_vec])
                    if with_dist and d8l == 15:
                        val = val + dist_v[s, pl.ds(jo, 16)]
                    boff = pl.multiple_of(
                        base_j + (d8l // 8) * 8192 + (d8l % 8) * 128, 16)
                    buf_v[bslot, pl.ds(boff, 16)] = val

        def start_out(bi, q, bslot):
            off = pl.multiple_of((base + bi) * (l * d) + q * QUARTER, QUARTER)
            pltpu.async_copy(buf_v.at[bslot],
                             out_hbm.at[pl.ds(off, QUARTER)], o_sems[bslot])

        def wait_out(bslot):
            pltpu.make_async_copy(buf_v.at[bslot],
                                  out_hbm.at[pl.ds(0, QUARTER)],
                                  o_sems[bslot]).wait()

        n_q = d // 16

        def b_body(i, _):
            for bb in range(2):
                bi = 2 * i + bb
                wait_fetch(bb)

                @pl.when(bi + 1 < b_per_w)
                def _():
                    start_fetch(bi + 1, 1 - bb)

                for q in range(n_q):
                    step = bb * n_q + q
                    bslot = step % 2
                    if step >= 2:
                        wait_out(bslot)
                    else:
                        @pl.when(i > 0)
                        def _():
                            wait_out(bslot)

                    quarter_compute(bb, q, bslot)
                    start_out(bi, q, bslot)
            return 0

        start_fetch(0, 0)
        lax.fori_loop(0, b_per_w // 2, b_body, 0)
        wait_out(0)
        wait_out(1)

    return run(tt, idx1, dist1)


def kernel(used_symbols, distribution, symbol_embeddings, positional_embeddings):
    b, l = used_symbols.shape
    v, dm1 = symbol_embeddings.shape
    d = dm1 + 1

    sym_padded = jnp.pad(symbol_embeddings, ((0, 0), (0, 1)))
    tt = _combine_tables(sym_padded, positional_embeddings)

    idx1 = used_symbols.astype(jnp.int32).reshape(b * l)
    dist1 = distribution.reshape(b * l)
    flat = _sc_embed(tt, idx1, dist1, b, l, d)

    out5 = flat.reshape(b, d // 8, l // 128, 8, 128)
    return out5.transpose(0, 2, 4, 1, 3).reshape(b, l, d)
